# baseline (speedup 1.0000x reference)
import functools

import jax
import jax.numpy as jnp
from jax.experimental import pallas as pl
from jax.experimental.pallas import tpu as pltpu

_B, _C, _L, _H, _W = 2, 192, 8, 64, 64
_E, _K, _P = 8, 2, 8
_nH, _nW = _H // _P, _W // _P
_N = _B * _nH * _nW
_S = _L * _P * _P
_RB = 8


def _router_body(x_ref, rw_ref, rb_ref, ids_ref, wts_ref):
    xb = x_ref[...].astype(jnp.float32)
    means = jnp.mean(xb, axis=1)
    logits = jnp.dot(means, rw_ref[...],
                     preferred_element_type=jnp.float32) + rb_ref[...]
    neg = jnp.finfo(jnp.float32).min
    v0 = jnp.max(logits, axis=1)
    i0 = jnp.argmax(logits, axis=1)
    lane = jax.lax.broadcasted_iota(jnp.int32, logits.shape, 1)
    l2 = jnp.where(lane == i0[:, None], neg, logits)
    v1 = jnp.max(l2, axis=1)
    i1 = jnp.argmax(l2, axis=1)
    e1 = jnp.exp(v1 - v0)
    w0 = 1.0 / (1.0 + e1)
    w1 = 1.0 - w0
    ids_ref[0] = jnp.concatenate([i0[:, None], i1[:, None]], axis=1)
    wts_ref[0] = jnp.concatenate([w0[:, None], w1[:, None]], axis=1)


def _expert_one(shifted, maskh_ref, dw_ref, lng_ref, lnb_ref,
                pwi_ref, pwib_ref, pwo_ref, w):
    acc = jnp.zeros((_S, _C), dtype=jnp.bfloat16)
    for ih in range(-3, 4):
        t = jnp.zeros((_S, _C), dtype=jnp.bfloat16)
        for j in range(-3, 4):
            tap = (ih + 3) * 7 + (j + 3)
            t = t + shifted[j + 3] * dw_ref[0, tap, :][None, :]
        t = pltpu.roll(t, shift=(-_P * ih) % _S, axis=0)
        acc = acc + t * maskh_ref[ih + 3]
    y = acc.astype(jnp.float32)

    yr = y.reshape(_L, _P * _P, _C)
    mu = jnp.mean(yr, axis=1, keepdims=True)
    var = jnp.mean(yr * yr, axis=1, keepdims=True) - mu * mu
    yn = (yr - mu) * jax.lax.rsqrt(var + 1e-5)
    yl = (yn * lng_ref[0][None] + lnb_ref[0][None]).reshape(_S, _C)

    h = jnp.dot(yl, pwi_ref[0],
                preferred_element_type=jnp.float32) + pwib_ref[0]
    a = h[:, :_C]
    gt = h[:, _C:]
    act = a * jax.nn.sigmoid(a) * gt
    return jnp.dot(act, pwo_ref[0] * w,
                   preferred_element_type=jnp.float32)


def _expert_body(ids_ref, x_ref, maskw_ref, maskh_ref,
                 dw0_ref, lng0_ref, lnb0_ref,
                 pwi0_ref, pwib0_ref, pwo0_ref, pwob0_ref,
                 dw1_ref, lng1_ref, lnb1_ref,
                 pwi1_ref, pwib1_ref, pwo1_ref, pwob1_ref,
                 wts_ref, out_ref):
    del ids_ref
    p = pl.program_id(0)
    w0 = wts_ref[p, 0]
    w1 = wts_ref[p, 1]

    xp = x_ref[0]

    shifted = []
    for j in range(-3, 4):
        src = pltpu.roll(xp, shift=(-j) % _S, axis=0)
        shifted.append(src * maskw_ref[j + 3])

    o0 = _expert_one(shifted, maskh_ref, dw0_ref, lng0_ref, lnb0_ref,
                     pwi0_ref, pwib0_ref, pwo0_ref, w0)
    o1 = _expert_one(shifted, maskh_ref, dw1_ref, lng1_ref, lnb1_ref,
                     pwi1_ref, pwib1_ref, pwo1_ref, w1)
    bias = w0 * pwob0_ref[0] + w1 * pwob1_ref[0]
    res = xp.astype(jnp.float32) + o0 + o1 + bias
    out_ref[0] = res.astype(jnp.bfloat16)


@jax.jit
def kernel(x, router_w, router_b, dw_w, dw_b, ln_g, ln_b,
           pwi_w, pwi_b, pwo_w, pwo_b):
    x_t = x.reshape(_B, _C, _L, _nH, _P, _nW, _P)
    x_t = x_t.transpose(0, 3, 5, 2, 4, 6, 1).reshape(_N, _S, _C)
    x_t = x_t.astype(jnp.bfloat16)

    rw_t = router_w.T
    rb_t = router_b.reshape(1, _E)
    dw_t = dw_w.reshape(_E, _C, 49).transpose(0, 2, 1).astype(jnp.bfloat16)
    del dw_b
    lng_t = jnp.broadcast_to(
        ln_g.reshape(_E, _P * _P, 1), (_E, _P * _P, _C))
    lnb_t = jnp.broadcast_to(
        ln_b.reshape(_E, _P * _P, 1), (_E, _P * _P, _C))
    pwi_t = pwi_w.transpose(0, 2, 1)
    pwib_t = pwi_b.reshape(_E, 1, 2 * _C)
    pwo_t = pwo_w.transpose(0, 2, 1)
    pwob_t = pwo_b.reshape(_E, 1, _C)

    nblk = _N // _RB
    ids, wts = pl.pallas_call(
        _router_body,
        grid=(nblk,),
        in_specs=[
            pl.BlockSpec((_RB, _S, _C), lambda i: (i, 0, 0)),
            pl.BlockSpec((_C, _E), lambda i: (0, 0)),
            pl.BlockSpec((1, _E), lambda i: (0, 0)),
        ],
        out_specs=[
            pl.BlockSpec((1, _RB, _K), lambda i: (i, 0, 0)),
            pl.BlockSpec((1, _RB, _K), lambda i: (i, 0, 0)),
        ],
        out_shape=[
            jax.ShapeDtypeStruct((nblk, _RB, _K), jnp.int32),
            jax.ShapeDtypeStruct((nblk, _RB, _K), jnp.float32),
        ],
    )(x_t, rw_t, rb_t)
    ids2 = ids.reshape(_N, _K)
    wts2 = wts.reshape(_N, _K)

    rows = jnp.arange(_S, dtype=jnp.int32)
    hh = (rows // _P) % _P
    ww = rows % _P
    offs = jnp.arange(-3, 4, dtype=jnp.int32)
    maskw = ((ww[None, :] + offs[:, None] >= 0)
             & (ww[None, :] + offs[:, None] < _P))
    maskh = ((hh[None, :] + offs[:, None] >= 0)
             & (hh[None, :] + offs[:, None] < _P))
    maskw_b = jnp.broadcast_to(
        maskw[:, :, None], (7, _S, _C)).astype(jnp.bfloat16)
    maskh_b = jnp.broadcast_to(
        maskh[:, :, None], (7, _S, _C)).astype(jnp.bfloat16)

    def _wspecs(k):
        return [
            pl.BlockSpec((1, 49, _C), lambda i, ids: (ids[i, k], 0, 0)),
            pl.BlockSpec((1, _P * _P, _C), lambda i, ids: (ids[i, k], 0, 0)),
            pl.BlockSpec((1, _P * _P, _C), lambda i, ids: (ids[i, k], 0, 0)),
            pl.BlockSpec((1, _C, 2 * _C), lambda i, ids: (ids[i, k], 0, 0)),
            pl.BlockSpec((1, 1, 2 * _C), lambda i, ids: (ids[i, k], 0, 0)),
            pl.BlockSpec((1, _C, _C), lambda i, ids: (ids[i, k], 0, 0)),
            pl.BlockSpec((1, 1, _C), lambda i, ids: (ids[i, k], 0, 0)),
        ]

    grid_spec = pltpu.PrefetchScalarGridSpec(
        num_scalar_prefetch=1,
        grid=(_N,),
        in_specs=(
            [pl.BlockSpec((1, _S, _C), lambda i, ids: (i, 0, 0)),
             pl.BlockSpec((7, _S, _C), lambda i, ids: (0, 0, 0)),
             pl.BlockSpec((7, _S, _C), lambda i, ids: (0, 0, 0))]
            + _wspecs(0) + _wspecs(1)
            + [pl.BlockSpec(memory_space=pltpu.SMEM)]
        ),
        out_specs=pl.BlockSpec((1, _S, _C), lambda i, ids: (i, 0, 0)),
    )
    wargs = (dw_t, lng_t, lnb_t, pwi_t, pwib_t, pwo_t, pwob_t)
    out_t = pl.pallas_call(
        _expert_body,
        grid_spec=grid_spec,
        out_shape=jax.ShapeDtypeStruct((_N, _S, _C), jnp.bfloat16),
    )(ids2, x_t, maskw_b, maskh_b, *wargs, *wargs, wts2)

    out = out_t.reshape(_B, _nH, _nW, _L, _P, _P, _C)
    out = out.transpose(0, 6, 3, 1, 4, 2, 5).reshape(_B, _C, _L, _H, _W)
    return out.astype(jnp.float32)


# speedup vs baseline: 1.0499x; 1.0499x over previous
import functools

import jax
import jax.numpy as jnp
from jax.experimental import pallas as pl
from jax.experimental.pallas import tpu as pltpu

_B, _C, _L, _H, _W = 2, 192, 8, 64, 64
_E, _K, _P = 8, 2, 8
_nH, _nW = _H // _P, _W // _P
_N = _B * _nH * _nW
_S = _L * _P * _P
_RB = 8


def _router_body(x_ref, rw_ref, rb_ref, ids_ref, wts_ref):
    xb = x_ref[...].astype(jnp.float32)
    means = jnp.mean(xb, axis=1)
    logits = jnp.dot(means, rw_ref[...],
                     preferred_element_type=jnp.float32) + rb_ref[...]
    neg = jnp.finfo(jnp.float32).min
    v0 = jnp.max(logits, axis=1)
    i0 = jnp.argmax(logits, axis=1)
    lane = jax.lax.broadcasted_iota(jnp.int32, logits.shape, 1)
    l2 = jnp.where(lane == i0[:, None], neg, logits)
    v1 = jnp.max(l2, axis=1)
    i1 = jnp.argmax(l2, axis=1)
    e1 = jnp.exp(v1 - v0)
    w0 = 1.0 / (1.0 + e1)
    w1 = 1.0 - w0
    ids_ref[0] = jnp.concatenate([i0[:, None], i1[:, None]], axis=1)
    wts_ref[0] = jnp.concatenate([w0[:, None], w1[:, None]], axis=1)


def _expert_one(shifted, maskh_ref, dw_ref, lng_ref, lnb_ref,
                pwi_ref, pwib_ref, pwo_ref, w):
    acc = jnp.zeros((_S, _C), dtype=jnp.bfloat16)
    for ih in range(-3, 4):
        t = jnp.zeros((_S, _C), dtype=jnp.bfloat16)
        for j in range(-3, 4):
            tap = (ih + 3) * 7 + (j + 3)
            t = t + shifted[j + 3] * dw_ref[0, tap, :][None, :]
        t = pltpu.roll(t, shift=(-_P * ih) % _S, axis=0)
        acc = acc + t * maskh_ref[ih + 3]
    y = acc.astype(jnp.float32)

    yr = y.reshape(_L, _P * _P, _C)
    mu = jnp.mean(yr, axis=1, keepdims=True)
    var = jnp.mean(yr * yr, axis=1, keepdims=True) - mu * mu
    yn = (yr - mu) * jax.lax.rsqrt(var + 1e-5)
    yl = (yn * lng_ref[0][None] + lnb_ref[0][None]).reshape(_S, _C)

    h = jnp.dot(yl, pwi_ref[0],
                preferred_element_type=jnp.float32) + pwib_ref[0]
    a = h[:, :_C]
    gt = h[:, _C:]
    act = a * jax.nn.sigmoid(a) * gt
    return jnp.dot(act, pwo_ref[0] * w,
                   preferred_element_type=jnp.float32)


def _expert_body(ids_ref, x_ref, maskw_ref, maskh_ref,
                 dw0_ref, lng0_ref, lnb0_ref,
                 pwi0_ref, pwib0_ref, pwo0_ref, pwob0_ref,
                 dw1_ref, lng1_ref, lnb1_ref,
                 pwi1_ref, pwib1_ref, pwo1_ref, pwob1_ref,
                 wts_ref, out_ref):
    del ids_ref
    p = pl.program_id(0)
    w0 = wts_ref[p, 0]
    w1 = wts_ref[p, 1]

    xp = x_ref[0]

    shifted = []
    for j in range(-3, 4):
        src = pltpu.roll(xp, shift=(-j) % _S, axis=0)
        shifted.append(src * maskw_ref[j + 3])

    o0 = _expert_one(shifted, maskh_ref, dw0_ref, lng0_ref, lnb0_ref,
                     pwi0_ref, pwib0_ref, pwo0_ref, w0)
    o1 = _expert_one(shifted, maskh_ref, dw1_ref, lng1_ref, lnb1_ref,
                     pwi1_ref, pwib1_ref, pwo1_ref, w1)
    bias = w0 * pwob0_ref[0] + w1 * pwob1_ref[0]
    out_ref[0] = xp.astype(jnp.float32) + o0 + o1 + bias


@jax.jit
def kernel(x, router_w, router_b, dw_w, dw_b, ln_g, ln_b,
           pwi_w, pwi_b, pwo_w, pwo_b):
    x_t = x.reshape(_B, _C, _L, _nH, _P, _nW, _P)
    x_t = x_t.transpose(0, 3, 5, 2, 4, 6, 1).reshape(_N, _S, _C)
    x_t = x_t.astype(jnp.bfloat16)

    rw_t = router_w.T
    rb_t = router_b.reshape(1, _E)
    dw_t = dw_w.reshape(_E, _C, 49).transpose(0, 2, 1).astype(jnp.bfloat16)
    del dw_b
    lng_t = jnp.broadcast_to(
        ln_g.reshape(_E, _P * _P, 1), (_E, _P * _P, _C))
    lnb_t = jnp.broadcast_to(
        ln_b.reshape(_E, _P * _P, 1), (_E, _P * _P, _C))
    pwi_t = pwi_w.transpose(0, 2, 1)
    pwib_t = pwi_b.reshape(_E, 1, 2 * _C)
    pwo_t = pwo_w.transpose(0, 2, 1)
    pwob_t = pwo_b.reshape(_E, 1, _C)

    nblk = _N // _RB
    ids, wts = pl.pallas_call(
        _router_body,
        grid=(nblk,),
        in_specs=[
            pl.BlockSpec((_RB, _S, _C), lambda i: (i, 0, 0)),
            pl.BlockSpec((_C, _E), lambda i: (0, 0)),
            pl.BlockSpec((1, _E), lambda i: (0, 0)),
        ],
        out_specs=[
            pl.BlockSpec((1, _RB, _K), lambda i: (i, 0, 0)),
            pl.BlockSpec((1, _RB, _K), lambda i: (i, 0, 0)),
        ],
        out_shape=[
            jax.ShapeDtypeStruct((nblk, _RB, _K), jnp.int32),
            jax.ShapeDtypeStruct((nblk, _RB, _K), jnp.float32),
        ],
    )(x_t, rw_t, rb_t)
    ids2 = ids.reshape(_N, _K)
    wts2 = wts.reshape(_N, _K)

    rows = jnp.arange(_S, dtype=jnp.int32)
    hh = (rows // _P) % _P
    ww = rows % _P
    offs = jnp.arange(-3, 4, dtype=jnp.int32)
    maskw = ((ww[None, :] + offs[:, None] >= 0)
             & (ww[None, :] + offs[:, None] < _P))
    maskh = ((hh[None, :] + offs[:, None] >= 0)
             & (hh[None, :] + offs[:, None] < _P))
    maskw_b = jnp.broadcast_to(
        maskw[:, :, None], (7, _S, _C)).astype(jnp.bfloat16)
    maskh_b = jnp.broadcast_to(
        maskh[:, :, None], (7, _S, _C)).astype(jnp.bfloat16)

    def _wspecs(k):
        return [
            pl.BlockSpec((1, 49, _C), lambda i, ids: (ids[i, k], 0, 0)),
            pl.BlockSpec((1, _P * _P, _C), lambda i, ids: (ids[i, k], 0, 0)),
            pl.BlockSpec((1, _P * _P, _C), lambda i, ids: (ids[i, k], 0, 0)),
            pl.BlockSpec((1, _C, 2 * _C), lambda i, ids: (ids[i, k], 0, 0)),
            pl.BlockSpec((1, 1, 2 * _C), lambda i, ids: (ids[i, k], 0, 0)),
            pl.BlockSpec((1, _C, _C), lambda i, ids: (ids[i, k], 0, 0)),
            pl.BlockSpec((1, 1, _C), lambda i, ids: (ids[i, k], 0, 0)),
        ]

    grid_spec = pltpu.PrefetchScalarGridSpec(
        num_scalar_prefetch=1,
        grid=(_N,),
        in_specs=(
            [pl.BlockSpec((1, _S, _C), lambda i, ids: (i, 0, 0)),
             pl.BlockSpec((7, _S, _C), lambda i, ids: (0, 0, 0)),
             pl.BlockSpec((7, _S, _C), lambda i, ids: (0, 0, 0))]
            + _wspecs(0) + _wspecs(1)
            + [pl.BlockSpec(memory_space=pltpu.SMEM)]
        ),
        out_specs=pl.BlockSpec((1, _S, _C), lambda i, ids: (i, 0, 0)),
    )
    wargs = (dw_t, lng_t, lnb_t, pwi_t, pwib_t, pwo_t, pwob_t)
    out_t = pl.pallas_call(
        _expert_body,
        grid_spec=grid_spec,
        out_shape=jax.ShapeDtypeStruct((_N, _S, _C), jnp.float32),
    )(ids2, x_t, maskw_b, maskh_b, *wargs, *wargs, wts2)

    out = out_t.reshape(_B, _nH, _nW, _L, _P, _P, _C)
    out = out.transpose(0, 6, 3, 1, 4, 2, 5).reshape(_B, _C, _L, _H, _W)
    return out
